# Initial kernel scaffold; baseline (speedup 1.0000x reference)
import jax
import jax.numpy as jnp
from jax.experimental import pallas as pl


def kernel(x, edge_index, W1, b1, W2, b2):
    raise NotImplementedError("write your pallas kernel here")


# speedup vs baseline: 6.9737x; 6.9737x over previous
import functools

import jax
import jax.numpy as jnp
from jax import lax
from jax.experimental import pallas as pl
from jax.experimental.pallas import tpu as pltpu
from jax.experimental.pallas import tpu_sc as plsc

N = 10000
E = 320000
D = 128
DH = D // 2
K = 3

NC = 2
NS = 16
NP = 10240
EP = 327680
CHUNK = 128
NCHUNKS = EP // CHUNK
GRP = 8
NG = NCHUNKS // GRP
GPT = NG // NS
RPT = NP // NS
RCH = 128
NRC = RPT // RCH

_MESH = plsc.VectorSubcoreMesh(core_axis_name="c", subcore_axis_name="s")


def _zero_sbuf(sbuf, width):
    def zrow(i, carry):
        for q in range(width // 16):
            sbuf[i, pl.ds(q * 16, 16)] = jnp.zeros((16,), jnp.float32)
        return carry
    lax.fori_loop(0, RCH, zrow, 0)


def _deg_body(rows_hbm, ones_hbm, deg0_hbm, deg1_hbm,
              idx8, onesv, sbuf, acc, sem):
    c = lax.axis_index("c")
    t = lax.axis_index("s")
    base = t * RPT

    _zero_sbuf(sbuf, 16)
    for k in range(NRC):
        pltpu.sync_copy(sbuf, acc.at[pl.ds(base + k * RCH, RCH)])
    pltpu.sync_copy(ones_hbm, onesv)
    plsc.subcore_barrier()

    gpt = NG // (NC * NS)
    g0 = (c * NS + t) * gpt

    def body(g, carry):
        pltpu.sync_copy(rows_hbm.at[pl.ds(g * GRP, GRP)], idx8)
        for r in range(GRP):
            pltpu.sync_copy(onesv, acc.at[idx8.at[r]], add=True)
        return carry
    lax.fori_loop(g0, g0 + gpt, body, 0)
    plsc.subcore_barrier()

    for k in range(NRC):
        s = pl.ds(base + k * RCH, RCH)
        pltpu.sync_copy(acc.at[s], sbuf)

        @pl.when(c == 0)
        def _():
            pltpu.sync_copy(sbuf, deg0_hbm.at[s])

        @pl.when(c == 1)
        def _():
            pltpu.sync_copy(sbuf, deg1_hbm.at[s])


_deg_call = pl.kernel(
    _deg_body,
    out_type=[jax.ShapeDtypeStruct((NP, 16), jnp.float32),
              jax.ShapeDtypeStruct((NP, 16), jnp.float32)],
    mesh=_MESH,
    compiler_params=pltpu.CompilerParams(use_tc_tiling_on_sc=False),
    scratch_types=[
        pltpu.VMEM((GRP, CHUNK), jnp.int32),
        pltpu.VMEM((CHUNK, 16), jnp.float32),
        pltpu.VMEM((RCH, 16), jnp.float32),
        pltpu.VMEM_SHARED((NP, 16), jnp.float32),
        pltpu.SemaphoreType.DMA,
    ],
)


def _layer_body(rows_hbm, cols_hbm, s0_hbm, s1_hbm, c_hbm,
                u1a_hbm, u1b_hbm, u2a_hbm, u2b_hbm,
                idx8r, idx8c, gbuf, sbuf, cv, acc1, acc2, sem):
    c = lax.axis_index("c")
    t = lax.axis_index("s")
    base = t * RPT

    _zero_sbuf(sbuf, DH)
    for k in range(NRC):
        pltpu.sync_copy(sbuf, acc1.at[pl.ds(base + k * RCH, RCH)])
        pltpu.sync_copy(sbuf, acc2.at[pl.ds(base + k * RCH, RCH)])
    plsc.subcore_barrier()

    g0 = t * GPT

    def spmm1(g, carry):
        pltpu.sync_copy(cols_hbm.at[pl.ds(g * GRP, GRP)], idx8c)
        pltpu.sync_copy(rows_hbm.at[pl.ds(g * GRP, GRP)], idx8r)
        for r in range(GRP):
            @pl.when(c == 0)
            def _():
                pltpu.async_copy(s0_hbm.at[idx8c.at[r]], gbuf, sem).wait()

            @pl.when(c == 1)
            def _():
                pltpu.async_copy(s1_hbm.at[idx8c.at[r]], gbuf, sem).wait()

            pltpu.sync_copy(gbuf, acc1.at[idx8r.at[r]], add=True)
        return carry
    lax.fori_loop(g0, g0 + GPT, spmm1, 0)
    plsc.subcore_barrier()

    for k in range(NRC):
        s = pl.ds(base + k * RCH, RCH)
        pltpu.sync_copy(acc1.at[s], sbuf)

        @pl.when(c == 0)
        def _():
            pltpu.sync_copy(sbuf, u1a_hbm.at[s])

        @pl.when(c == 1)
        def _():
            pltpu.sync_copy(sbuf, u1b_hbm.at[s])

        pltpu.sync_copy(c_hbm.at[s], cv)

        def srow(i, carry):
            for q in range(DH // 16):
                sl = (i, pl.ds(q * 16, 16))
                sbuf[sl] = sbuf[sl] * cv[sl]
            return carry
        lax.fori_loop(0, RCH, srow, 0)
        pltpu.sync_copy(sbuf, acc1.at[s])
    plsc.subcore_barrier()

    def spmm2(g, carry):
        pltpu.sync_copy(cols_hbm.at[pl.ds(g * GRP, GRP)], idx8c)
        pltpu.sync_copy(rows_hbm.at[pl.ds(g * GRP, GRP)], idx8r)
        for r in range(GRP):
            pltpu.async_copy(acc1.at[idx8c.at[r]], gbuf, sem).wait()
            pltpu.sync_copy(gbuf, acc2.at[idx8r.at[r]], add=True)
        return carry
    lax.fori_loop(g0, g0 + GPT, spmm2, 0)
    plsc.subcore_barrier()

    for k in range(NRC):
        s = pl.ds(base + k * RCH, RCH)
        pltpu.sync_copy(acc2.at[s], sbuf)

        @pl.when(c == 0)
        def _():
            pltpu.sync_copy(sbuf, u2a_hbm.at[s])

        @pl.when(c == 1)
        def _():
            pltpu.sync_copy(sbuf, u2b_hbm.at[s])


_layer_call = pl.kernel(
    _layer_body,
    out_type=[jax.ShapeDtypeStruct((NP, DH), jnp.float32)] * 4,
    mesh=_MESH,
    compiler_params=pltpu.CompilerParams(use_tc_tiling_on_sc=False),
    scratch_types=[
        pltpu.VMEM((GRP, CHUNK), jnp.int32),
        pltpu.VMEM((GRP, CHUNK), jnp.int32),
        pltpu.VMEM((CHUNK, DH), jnp.float32),
        pltpu.VMEM((RCH, DH), jnp.float32),
        pltpu.VMEM((RCH, DH), jnp.float32),
        pltpu.VMEM_SHARED((NP, DH), jnp.float32),
        pltpu.VMEM_SHARED((NP, DH), jnp.float32),
        pltpu.SemaphoreType.DMA,
    ],
)


BM = 640


def _prep_body(d0_ref, d1_ref, x_ref, dis_o, c_o, sa_o, sb_o):
    deg = d0_ref[:, 0:1] + d1_ref[:, 0:1]
    dis = jnp.where(deg > 0, lax.rsqrt(jnp.maximum(deg, 1.0)), 0.0)
    dis_o[...] = dis
    c_o[...] = jnp.broadcast_to(-(dis * dis), (BM, DH))
    s = x_ref[...] * dis
    sa_o[...] = s[:, :DH]
    sb_o[...] = s[:, DH:]


def _prep(d0, d1, x):
    return pl.pallas_call(
        _prep_body,
        grid=(NP // BM,),
        in_specs=[
            pl.BlockSpec((BM, 16), lambda i: (i, 0)),
            pl.BlockSpec((BM, 16), lambda i: (i, 0)),
            pl.BlockSpec((BM, D), lambda i: (i, 0)),
        ],
        out_specs=[
            pl.BlockSpec((BM, 1), lambda i: (i, 0)),
            pl.BlockSpec((BM, DH), lambda i: (i, 0)),
            pl.BlockSpec((BM, DH), lambda i: (i, 0)),
            pl.BlockSpec((BM, DH), lambda i: (i, 0)),
        ],
        out_shape=[
            jax.ShapeDtypeStruct((NP, 1), jnp.float32),
            jax.ShapeDtypeStruct((NP, DH), jnp.float32),
            jax.ShapeDtypeStruct((NP, DH), jnp.float32),
            jax.ShapeDtypeStruct((NP, DH), jnp.float32),
        ],
    )(d0, d1, x)


def _combine_body(t_ref, u1a, u1b, u2a, u2b, dis_ref, w_ref, b_ref, *out_refs,
                  relu):
    dis = dis_ref[...]
    tt = t_ref[...]
    u1 = jnp.concatenate([u1a[...], u1b[...]], axis=1)
    u2 = jnp.concatenate([u2a[...], u2b[...]], axis=1)
    tx1 = -dis * u1
    tx2 = -2.0 * dis * u2 - tt
    w = w_ref[...]
    acc = jnp.dot(tt, w[0], preferred_element_type=jnp.float32)
    acc = acc + jnp.dot(tx1, w[1], preferred_element_type=jnp.float32)
    acc = acc + jnp.dot(tx2, w[2], preferred_element_type=jnp.float32)
    acc = acc + b_ref[...]
    if relu:
        h = jnp.maximum(acc, 0.0)
        out_refs[0][...] = h
        s = h * dis
        out_refs[1][...] = s[:, :DH]
        out_refs[2][...] = s[:, DH:]
    else:
        out_refs[0][...] = acc


def _combine(t, u1a, u1b, u2a, u2b, dis, w, b, relu):
    if relu:
        out_shape = [
            jax.ShapeDtypeStruct((NP, D), jnp.float32),
            jax.ShapeDtypeStruct((NP, DH), jnp.float32),
            jax.ShapeDtypeStruct((NP, DH), jnp.float32),
        ]
        out_specs = [
            pl.BlockSpec((BM, D), lambda i: (i, 0)),
            pl.BlockSpec((BM, DH), lambda i: (i, 0)),
            pl.BlockSpec((BM, DH), lambda i: (i, 0)),
        ]
    else:
        out_shape = [jax.ShapeDtypeStruct((NP, D), jnp.float32)]
        out_specs = [pl.BlockSpec((BM, D), lambda i: (i, 0))]
    return pl.pallas_call(
        functools.partial(_combine_body, relu=relu),
        grid=(NP // BM,),
        in_specs=[
            pl.BlockSpec((BM, D), lambda i: (i, 0)),
            pl.BlockSpec((BM, DH), lambda i: (i, 0)),
            pl.BlockSpec((BM, DH), lambda i: (i, 0)),
            pl.BlockSpec((BM, DH), lambda i: (i, 0)),
            pl.BlockSpec((BM, DH), lambda i: (i, 0)),
            pl.BlockSpec((BM, 1), lambda i: (i, 0)),
            pl.BlockSpec((K, D, D), lambda i: (0, 0, 0)),
            pl.BlockSpec((1, D), lambda i: (0, 0)),
        ],
        out_specs=out_specs,
        out_shape=out_shape,
    )(t, u1a, u1b, u2a, u2b, dis, w, b)


def kernel(x, edge_index, W1, b1, W2, b2):
    ei = edge_index.astype(jnp.int32)
    pad = jnp.full((2, EP - E), N, jnp.int32)
    ei = jnp.concatenate([ei, pad], axis=1)
    rows = ei[0].reshape(NCHUNKS, CHUNK)
    cols = ei[1].reshape(NCHUNKS, CHUNK)
    ones16 = jnp.ones((CHUNK, 16), jnp.float32)
    xp = jnp.pad(x, ((0, NP - N), (0, 0)))

    d0, d1 = _deg_call(rows, ones16)
    dis, cmat, sa, sb = _prep(d0, d1, xp)

    u1a, u1b, u2a, u2b = _layer_call(rows, cols, sa, sb, cmat)
    h, sa1, sb1 = _combine(xp, u1a, u1b, u2a, u2b, dis, W1,
                           b1.reshape(1, D), relu=True)
    v1a, v1b, v2a, v2b = _layer_call(rows, cols, sa1, sb1, cmat)
    (out,) = _combine(h, v1a, v1b, v2a, v2b, dis, W2,
                      b2.reshape(1, D), relu=False)
    return out[:N]


# speedup vs baseline: 10.0702x; 1.4440x over previous
import functools

import jax
import jax.numpy as jnp
from jax import lax
from jax.experimental import pallas as pl
from jax.experimental.pallas import tpu as pltpu
from jax.experimental.pallas import tpu_sc as plsc

N = 10000
E = 320000
D = 128
DH = D // 2
K = 3

NC = 2
NS = 16
NP = 10240
EP = 327680
CHUNK = 128
NCHUNKS = EP // CHUNK
GRP = 8
NG = NCHUNKS // GRP
GPT = NG // NS
RPT = NP // NS
RCH = 128
NRC = RPT // RCH

_MESH = plsc.VectorSubcoreMesh(core_axis_name="c", subcore_axis_name="s")


def _zero_sbuf(sbuf, width):
    def zrow(i, carry):
        for q in range(width // 16):
            sbuf[i, pl.ds(q * 16, 16)] = jnp.zeros((16,), jnp.float32)
        return carry
    lax.fori_loop(0, RCH, zrow, 0)


def _deg_body(rows_hbm, ones_hbm, deg0_hbm, deg1_hbm,
              idx8, onesv, sbuf, acc, sem):
    c = lax.axis_index("c")
    t = lax.axis_index("s")
    base = t * RPT

    _zero_sbuf(sbuf, 16)
    for k in range(NRC):
        pltpu.sync_copy(sbuf, acc.at[pl.ds(base + k * RCH, RCH)])
    pltpu.sync_copy(ones_hbm, onesv)
    plsc.subcore_barrier()

    gpt = NG // (NC * NS)
    g0 = (c * NS + t) * gpt

    def body(g, carry):
        pltpu.sync_copy(rows_hbm.at[pl.ds(g * GRP, GRP)], idx8)
        for r in range(GRP):
            pltpu.sync_copy(onesv, acc.at[idx8.at[r]], add=True)
        return carry
    lax.fori_loop(g0, g0 + gpt, body, 0)
    plsc.subcore_barrier()

    for k in range(NRC):
        s = pl.ds(base + k * RCH, RCH)
        pltpu.sync_copy(acc.at[s], sbuf)

        @pl.when(c == 0)
        def _():
            pltpu.sync_copy(sbuf, deg0_hbm.at[s])

        @pl.when(c == 1)
        def _():
            pltpu.sync_copy(sbuf, deg1_hbm.at[s])


_deg_call = pl.kernel(
    _deg_body,
    out_type=[jax.ShapeDtypeStruct((NP, 16), jnp.float32),
              jax.ShapeDtypeStruct((NP, 16), jnp.float32)],
    mesh=_MESH,
    compiler_params=pltpu.CompilerParams(use_tc_tiling_on_sc=False),
    scratch_types=[
        pltpu.VMEM((GRP, CHUNK), jnp.int32),
        pltpu.VMEM((CHUNK, 16), jnp.float32),
        pltpu.VMEM((RCH, 16), jnp.float32),
        pltpu.VMEM_SHARED((NP, 16), jnp.float32),
        pltpu.SemaphoreType.DMA,
    ],
)


NCT = NCHUNKS // NS
PG = 2
NGRP = NCT // PG


def _pipelined_spmm(t, rows_hbm, cols_hbm, src, acc,
                    rbufs, cbufs, sets, gsems, ssems, isr, isc):
    g0 = t * NGRP

    def cslice(g):
        return cols_hbm.at[pl.ds((g0 + g) * PG, PG)]

    def rslice(g):
        return rows_hbm.at[pl.ds((g0 + g) * PG, PG)]

    pltpu.async_copy(cslice(0), cbufs[0], isc[0])
    pltpu.async_copy(rslice(0), rbufs[0], isr[0])
    pltpu.async_copy(cslice(1), cbufs[1], isc[1])
    pltpu.make_async_copy(cslice(0), cbufs[0], isc[0]).wait()
    for r in range(PG):
        pltpu.async_copy(src.at[cbufs[0].at[r]], sets[0][r], gsems[0])

    def body(g, carry):
        for s in range(2):
            nx = 1 - s

            @pl.when(g % 2 == s)
            def _(s=s, nx=nx):
                @pl.when(g > 0)
                def _():
                    for r in range(PG):
                        pltpu.make_async_copy(
                            sets[nx][r], acc.at[rbufs[nx].at[r]],
                            ssems[nx]).wait()

                @pl.when(g < NGRP - 1)
                def _():
                    pltpu.async_copy(rslice(g + 1), rbufs[nx], isr[nx])
                    pltpu.make_async_copy(cslice(g + 1), cbufs[nx],
                                          isc[nx]).wait()
                    for r in range(PG):
                        pltpu.async_copy(src.at[cbufs[nx].at[r]], sets[nx][r],
                                         gsems[nx])

                for r in range(PG):
                    pltpu.make_async_copy(src.at[cbufs[s].at[r]], sets[s][r],
                                          gsems[s]).wait()

                @pl.when(g < NGRP - 2)
                def _():
                    pltpu.async_copy(cslice(g + 2), cbufs[s], isc[s])

                pltpu.make_async_copy(rslice(g), rbufs[s], isr[s]).wait()
                for r in range(PG):
                    pltpu.async_copy(sets[s][r], acc.at[rbufs[s].at[r]],
                                     ssems[s], add=True)
        return carry
    lax.fori_loop(0, NGRP, body, 0)

    last = (NGRP - 1) % 2
    for r in range(PG):
        pltpu.make_async_copy(sets[last][r], acc.at[rbufs[last].at[r]],
                              ssems[last]).wait()


def _layer_body(rows_hbm, cols_hbm, s0_hbm, s1_hbm, c_hbm,
                u1a_hbm, u1b_hbm, u2a_hbm, u2b_hbm,
                rb0, rb1, cb0, cb1, b0, b1, b2, b3, acc1, acc2,
                gsem0, gsem1, ssem0, ssem1, isr0, isr1, isc0, isc1):
    c = lax.axis_index("c")
    t = lax.axis_index("s")
    base = t * RPT
    rbufs = (rb0, rb1)
    cbufs = (cb0, cb1)
    sets = ((b0, b1), (b2, b3))
    gsems = (gsem0, gsem1)
    ssems = (ssem0, ssem1)
    isr = (isr0, isr1)
    isc = (isc0, isc1)
    sbuf, cv = b0, b1

    _zero_sbuf(sbuf, DH)
    for k in range(NRC):
        pltpu.sync_copy(sbuf, acc1.at[pl.ds(base + k * RCH, RCH)])
        pltpu.sync_copy(sbuf, acc2.at[pl.ds(base + k * RCH, RCH)])
    plsc.subcore_barrier()

    @pl.when(c == 0)
    def _():
        _pipelined_spmm(t, rows_hbm, cols_hbm, s0_hbm, acc1,
                        rbufs, cbufs, sets, gsems, ssems, isr, isc)

    @pl.when(c == 1)
    def _():
        _pipelined_spmm(t, rows_hbm, cols_hbm, s1_hbm, acc1,
                        rbufs, cbufs, sets, gsems, ssems, isr, isc)

    plsc.subcore_barrier()

    for k in range(NRC):
        s = pl.ds(base + k * RCH, RCH)
        pltpu.sync_copy(acc1.at[s], sbuf)

        @pl.when(c == 0)
        def _():
            pltpu.sync_copy(sbuf, u1a_hbm.at[s])

        @pl.when(c == 1)
        def _():
            pltpu.sync_copy(sbuf, u1b_hbm.at[s])

        pltpu.sync_copy(c_hbm.at[s], cv)

        def srow(i, carry):
            for q in range(DH // 16):
                sl = (i, pl.ds(q * 16, 16))
                sbuf[sl] = sbuf[sl] * cv[sl]
            return carry
        lax.fori_loop(0, RCH, srow, 0)
        pltpu.sync_copy(sbuf, acc1.at[s])
    plsc.subcore_barrier()

    _pipelined_spmm(t, rows_hbm, cols_hbm, acc1, acc2,
                    rbufs, cbufs, sets, gsems, ssems, isr, isc)
    plsc.subcore_barrier()

    for k in range(NRC):
        s = pl.ds(base + k * RCH, RCH)
        pltpu.sync_copy(acc2.at[s], sbuf)

        @pl.when(c == 0)
        def _():
            pltpu.sync_copy(sbuf, u2a_hbm.at[s])

        @pl.when(c == 1)
        def _():
            pltpu.sync_copy(sbuf, u2b_hbm.at[s])


_layer_call = pl.kernel(
    _layer_body,
    out_type=[jax.ShapeDtypeStruct((NP, DH), jnp.float32)] * 4,
    mesh=_MESH,
    compiler_params=pltpu.CompilerParams(use_tc_tiling_on_sc=False),
    scratch_types=[
        pltpu.VMEM((PG, CHUNK), jnp.int32),
        pltpu.VMEM((PG, CHUNK), jnp.int32),
        pltpu.VMEM((PG, CHUNK), jnp.int32),
        pltpu.VMEM((PG, CHUNK), jnp.int32),
        pltpu.VMEM((CHUNK, DH), jnp.float32),
        pltpu.VMEM((CHUNK, DH), jnp.float32),
        pltpu.VMEM((CHUNK, DH), jnp.float32),
        pltpu.VMEM((CHUNK, DH), jnp.float32),
        pltpu.VMEM_SHARED((NP, DH), jnp.float32),
        pltpu.VMEM_SHARED((NP, DH), jnp.float32),
    ] + [pltpu.SemaphoreType.DMA] * 8,
)


BM = 640


def _prep_body(d0_ref, d1_ref, x_ref, dis_o, c_o, sa_o, sb_o):
    deg = d0_ref[:, 0:1] + d1_ref[:, 0:1]
    dis = jnp.where(deg > 0, lax.rsqrt(jnp.maximum(deg, 1.0)), 0.0)
    dis_o[...] = dis
    c_o[...] = jnp.broadcast_to(-(dis * dis), (BM, DH))
    s = x_ref[...] * dis
    sa_o[...] = s[:, :DH]
    sb_o[...] = s[:, DH:]


def _prep(d0, d1, x):
    return pl.pallas_call(
        _prep_body,
        grid=(NP // BM,),
        in_specs=[
            pl.BlockSpec((BM, 16), lambda i: (i, 0)),
            pl.BlockSpec((BM, 16), lambda i: (i, 0)),
            pl.BlockSpec((BM, D), lambda i: (i, 0)),
        ],
        out_specs=[
            pl.BlockSpec((BM, 1), lambda i: (i, 0)),
            pl.BlockSpec((BM, DH), lambda i: (i, 0)),
            pl.BlockSpec((BM, DH), lambda i: (i, 0)),
            pl.BlockSpec((BM, DH), lambda i: (i, 0)),
        ],
        out_shape=[
            jax.ShapeDtypeStruct((NP, 1), jnp.float32),
            jax.ShapeDtypeStruct((NP, DH), jnp.float32),
            jax.ShapeDtypeStruct((NP, DH), jnp.float32),
            jax.ShapeDtypeStruct((NP, DH), jnp.float32),
        ],
    )(d0, d1, x)


def _combine_body(t_ref, u1a, u1b, u2a, u2b, dis_ref, w_ref, b_ref, *out_refs,
                  relu):
    dis = dis_ref[...]
    tt = t_ref[...]
    u1 = jnp.concatenate([u1a[...], u1b[...]], axis=1)
    u2 = jnp.concatenate([u2a[...], u2b[...]], axis=1)
    tx1 = -dis * u1
    tx2 = -2.0 * dis * u2 - tt
    w = w_ref[...]
    acc = jnp.dot(tt, w[0], preferred_element_type=jnp.float32)
    acc = acc + jnp.dot(tx1, w[1], preferred_element_type=jnp.float32)
    acc = acc + jnp.dot(tx2, w[2], preferred_element_type=jnp.float32)
    acc = acc + b_ref[...]
    if relu:
        h = jnp.maximum(acc, 0.0)
        out_refs[0][...] = h
        s = h * dis
        out_refs[1][...] = s[:, :DH]
        out_refs[2][...] = s[:, DH:]
    else:
        out_refs[0][...] = acc


def _combine(t, u1a, u1b, u2a, u2b, dis, w, b, relu):
    if relu:
        out_shape = [
            jax.ShapeDtypeStruct((NP, D), jnp.float32),
            jax.ShapeDtypeStruct((NP, DH), jnp.float32),
            jax.ShapeDtypeStruct((NP, DH), jnp.float32),
        ]
        out_specs = [
            pl.BlockSpec((BM, D), lambda i: (i, 0)),
            pl.BlockSpec((BM, DH), lambda i: (i, 0)),
            pl.BlockSpec((BM, DH), lambda i: (i, 0)),
        ]
    else:
        out_shape = [jax.ShapeDtypeStruct((NP, D), jnp.float32)]
        out_specs = [pl.BlockSpec((BM, D), lambda i: (i, 0))]
    return pl.pallas_call(
        functools.partial(_combine_body, relu=relu),
        grid=(NP // BM,),
        in_specs=[
            pl.BlockSpec((BM, D), lambda i: (i, 0)),
            pl.BlockSpec((BM, DH), lambda i: (i, 0)),
            pl.BlockSpec((BM, DH), lambda i: (i, 0)),
            pl.BlockSpec((BM, DH), lambda i: (i, 0)),
            pl.BlockSpec((BM, DH), lambda i: (i, 0)),
            pl.BlockSpec((BM, 1), lambda i: (i, 0)),
            pl.BlockSpec((K, D, D), lambda i: (0, 0, 0)),
            pl.BlockSpec((1, D), lambda i: (0, 0)),
        ],
        out_specs=out_specs,
        out_shape=out_shape,
    )(t, u1a, u1b, u2a, u2b, dis, w, b)


def kernel(x, edge_index, W1, b1, W2, b2):
    ei = edge_index.astype(jnp.int32)
    pad = jnp.full((2, EP - E), N, jnp.int32)
    ei = jnp.concatenate([ei, pad], axis=1)
    rows = ei[0].reshape(NCHUNKS, CHUNK)
    cols = ei[1].reshape(NCHUNKS, CHUNK)
    ones16 = jnp.ones((CHUNK, 16), jnp.float32)
    xp = jnp.pad(x, ((0, NP - N), (0, 0)))

    d0, d1 = _deg_call(rows, ones16)
    dis, cmat, sa, sb = _prep(d0, d1, xp)

    u1a, u1b, u2a, u2b = _layer_call(rows, cols, sa, sb, cmat)
    h, sa1, sb1 = _combine(xp, u1a, u1b, u2a, u2b, dis, W1,
                           b1.reshape(1, D), relu=True)
    v1a, v1b, v2a, v2b = _layer_call(rows, cols, sa1, sb1, cmat)
    (out,) = _combine(h, v1a, v1b, v2a, v2b, dis, W2,
                      b2.reshape(1, D), relu=False)
    return out[:N]
